# baseline (speedup 1.0000x reference)
import functools

import jax
import jax.numpy as jnp
from jax.experimental import pallas as pl
from jax.experimental.pallas import tpu as pltpu


def _se_step(x_ref, w1_ref, w2_ref, o_ref, *, inv_hw):
    xb = x_ref[...]
    pooled = jnp.sum(xb, axis=-1) * inv_hw
    h = jax.lax.dot_general(pooled, w1_ref[...], (((1,), (1,)), ((), ())),
                            preferred_element_type=jnp.float32)
    h = jnp.maximum(h, 0.0)
    logits = jax.lax.dot_general(h, w2_ref[...], (((1,), (1,)), ((), ())),
                                 preferred_element_type=jnp.float32)
    gate = jax.nn.sigmoid(logits)
    o_ref[...] = xb * gate[:, :, None]


def _pick_tb(B, per_batch_bytes):
    budget = 12 << 20
    tb = 1
    for d in range(1, B + 1):
        if B % d == 0 and d * per_batch_bytes <= budget and (B // d) % 2 == 0:
            tb = d
    return tb


def kernel(x, w1, w2):
    B, C, H, W = x.shape
    HW = H * W
    x3 = x.reshape(B, C, HW)
    TB = _pick_tb(B, C * HW * x.dtype.itemsize)
    out = pl.pallas_call(
        functools.partial(_se_step, inv_hw=1.0 / float(HW)),
        out_shape=jax.ShapeDtypeStruct((B, C, HW), x.dtype),
        grid=(B // TB,),
        in_specs=[
            pl.BlockSpec((TB, C, HW), lambda b: (b, 0, 0)),
            pl.BlockSpec(w1.shape, lambda b: (0, 0)),
            pl.BlockSpec(w2.shape, lambda b: (0, 0)),
        ],
        out_specs=pl.BlockSpec((TB, C, HW), lambda b: (b, 0, 0)),
        compiler_params=pltpu.CompilerParams(
            dimension_semantics=("parallel",),
            vmem_limit_bytes=56 << 20,
        ),
    )(x3, w1, w2)
    return out.reshape(B, C, H, W)
